# Initial kernel scaffold; baseline (speedup 1.0000x reference)
import jax
import jax.numpy as jnp
from jax.experimental import pallas as pl


def kernel(x, edge_index, W1l, b1l, W1r, W2l, b2l, W2r, Wo, bo):
    raise NotImplementedError("write your pallas kernel here")


# speedup vs baseline: 4.5513x; 4.5513x over previous
import dataclasses
import functools

import jax
import jax.numpy as jnp
from jax import lax
from jax.experimental import pallas as pl
from jax.experimental.pallas import tpu as pltpu
from jax.experimental.pallas import tpu_sc as plsc

N = 10000
E = 320000
D = 128
DH = 64
NC = 2
NS = 16
NW = NC * NS
STREAM = 128
RPS = 158
EP = NS * RPS * STREAM
ACC_N = 10112
STRIPE = ACC_N // NS
TRASH = N
BLK = 2000
FP = jax.lax.Precision.HIGHEST


def _sc_agg(xh, packed):
    mesh = plsc.VectorSubcoreMesh(core_axis_name="c", subcore_axis_name="s")
    cp = pltpu.CompilerParams(use_tc_tiling_on_sc=False)
    if "needs_layout_passes" in pltpu.CompilerParams.__dataclass_fields__:
        cp = dataclasses.replace(cp, needs_layout_passes=False)

    @functools.partial(
        pl.kernel,
        mesh=mesh,
        compiler_params=cp,
        out_type=(
            jax.ShapeDtypeStruct((NC, ACC_N, DH), jnp.float32),
            jax.ShapeDtypeStruct((NS, ACC_N), jnp.float32),
        ),
        scratch_types=[
            pltpu.VMEM((RPS, STREAM), jnp.int32),
            pltpu.VMEM((RPS, STREAM), jnp.int32),
            pltpu.VMEM((RPS, STREAM), jnp.int32),
            pltpu.VMEM((STREAM, DH), jnp.float32),
            pltpu.VMEM((STRIPE // 4, DH), jnp.float32),
            pltpu.VMEM((ACC_N,), jnp.float32),
            pltpu.VMEM_SHARED((ACC_N, DH), jnp.float32),
            pltpu.SemaphoreType.DMA,
        ],
    )
    def k(xh_hbm, pk_hbm, p_out, deg_out, pbuf, sidx, didx, rows, zbuf,
          degv, acc, sem):
        c = lax.axis_index("c")
        s = lax.axis_index("s")

        pltpu.sync_copy(pk_hbm.at[s], pbuf)

        @pl.loop(0, RPS)
        def _(j):
            @pl.loop(0, STREAM, step=16)
            def _(kk):
                v = pbuf[j, pl.ds(kk, 16)]
                sidx[j, pl.ds(kk, 16)] = lax.bitwise_and(v, 16383)
                didx[j, pl.ds(kk, 16)] = lax.shift_right_logical(v, 14)

        @pl.loop(0, STRIPE // 4)
        def _(r):
            @pl.loop(0, DH, step=16)
            def _(jj):
                zbuf[r, pl.ds(jj, 16)] = jnp.zeros((16,), jnp.float32)

        for kk in range(4):
            pltpu.sync_copy(
                zbuf, acc.at[pl.ds(s * STRIPE + kk * (STRIPE // 4), STRIPE // 4)])

        @pl.loop(0, ACC_N, step=16)
        def _(i):
            degv[pl.ds(i, 16)] = jnp.zeros((16,), jnp.float32)

        plsc.subcore_barrier()

        ones16 = jnp.ones((16,), jnp.float32)

        @pl.loop(0, RPS)
        def _(j):
            pltpu.async_copy(xh_hbm.at[c].at[sidx.at[j]], rows, sem).wait()
            pltpu.sync_copy(rows, acc.at[didx.at[j]], add=True)

        @pl.when(c == 0)
        def _():
            @pl.loop(0, RPS)
            def _(j):
                drow = didx.at[j]

                @pl.loop(0, STREAM, step=16)
                def _(kk):
                    plsc.addupdate_scatter(degv, [drow[pl.ds(kk, 16)]], ones16)

            pltpu.sync_copy(degv, deg_out.at[s])

        plsc.subcore_barrier()

        pltpu.sync_copy(
            acc.at[pl.ds(s * STRIPE, STRIPE)],
            p_out.at[c].at[pl.ds(s * STRIPE, STRIPE)],
        )

    return k(xh, packed)


def _layer_body(p0_ref, p1_ref, dg_ref, x_ref, wl_ref, bl_ref, wr_ref, out_ref):
    accf = jnp.concatenate([p0_ref[0], p1_ref[0]], axis=1)
    deg = jnp.maximum(jnp.sum(dg_ref[...], axis=1, keepdims=True), 1.0)
    agg = accf / deg
    h = (
        jnp.dot(agg, wl_ref[...], precision=FP)
        + bl_ref[...]
        + jnp.dot(x_ref[...], wr_ref[...], precision=FP)
    )
    out_ref[...] = jnp.maximum(h, 0.0)


def _final_body(p0_ref, p1_ref, dg_ref, h_ref, wl_ref, bl_ref, wr_ref,
                wo_ref, bo_ref, out_ref):
    accf = jnp.concatenate([p0_ref[0], p1_ref[0]], axis=1)
    deg = jnp.maximum(jnp.sum(dg_ref[...], axis=1, keepdims=True), 1.0)
    agg = accf / deg
    h = (
        jnp.dot(agg, wl_ref[...], precision=FP)
        + bl_ref[...]
        + jnp.dot(h_ref[...], wr_ref[...], precision=FP)
    )
    h = jnp.maximum(h, 0.0)
    wod = wo_ref[1:2, :] - wo_ref[0:1, :]
    ld = jnp.sum(h * wod, axis=1, keepdims=True)
    ld = ld + (bo_ref[1] - bo_ref[0])
    out_ref[...] = 1.0 / (1.0 + jnp.exp(-ld))


def _layer_call(p, degT, x, wl, bl, wr):
    return pl.pallas_call(
        _layer_body,
        grid=(N // BLK,),
        in_specs=[
            pl.BlockSpec((1, BLK, DH), lambda i: (0, i, 0)),
            pl.BlockSpec((1, BLK, DH), lambda i: (1, i, 0)),
            pl.BlockSpec((BLK, NS), lambda i: (i, 0)),
            pl.BlockSpec((BLK, D), lambda i: (i, 0)),
            pl.BlockSpec((D, D), lambda i: (0, 0)),
            pl.BlockSpec((1, D), lambda i: (0, 0)),
            pl.BlockSpec((D, D), lambda i: (0, 0)),
        ],
        out_specs=pl.BlockSpec((BLK, D), lambda i: (i, 0)),
        out_shape=jax.ShapeDtypeStruct((N, D), jnp.float32),
    )(p, p, degT, x, wl, bl, wr)


def _final_call(p, degT, h, wl, bl, wr, woT, bo2):
    return pl.pallas_call(
        _final_body,
        grid=(N // BLK,),
        in_specs=[
            pl.BlockSpec((1, BLK, DH), lambda i: (0, i, 0)),
            pl.BlockSpec((1, BLK, DH), lambda i: (1, i, 0)),
            pl.BlockSpec((BLK, NS), lambda i: (i, 0)),
            pl.BlockSpec((BLK, D), lambda i: (i, 0)),
            pl.BlockSpec((D, D), lambda i: (0, 0)),
            pl.BlockSpec((1, D), lambda i: (0, 0)),
            pl.BlockSpec((D, D), lambda i: (0, 0)),
            pl.BlockSpec((2, D), lambda i: (0, 0)),
            pl.BlockSpec(memory_space=pltpu.SMEM),
        ],
        out_specs=pl.BlockSpec((BLK, 1), lambda i: (i, 0)),
        out_shape=jax.ShapeDtypeStruct((N, 1), jnp.float32),
    )(p, p, degT, h, wl, bl, wr, woT, bo2)


def _split_cols(x):
    return jnp.stack([x[:, :DH], x[:, DH:]])


def kernel(x, edge_index, W1l, b1l, W1r, W2l, b2l, W2r, Wo, bo):
    ei = edge_index.astype(jnp.int32)
    npad = EP - E
    src_p = jnp.concatenate([ei[0], jnp.zeros((npad,), jnp.int32)])
    dst_p = jnp.concatenate([ei[1], jnp.full((npad,), TRASH, jnp.int32)])
    packed = ((dst_p << 14) | src_p).reshape(NS, RPS, STREAM)

    b1 = b1l.reshape(1, D)
    b2 = b2l.reshape(1, D)
    woT = Wo.T
    bo2 = bo

    p1, degs = _sc_agg(_split_cols(x), packed)
    degT = degs.T[:N]
    h1 = _layer_call(p1[:, :N], degT, x, W1l, b1, W1r)
    p2, _ = _sc_agg(_split_cols(h1), packed)
    out = _final_call(p2[:, :N], degT, h1, W2l, b2, W2r, woT, bo2)
    return out.reshape(N)


# speedup vs baseline: 5.4492x; 1.1973x over previous
import dataclasses
import functools

import jax
import jax.numpy as jnp
from jax import lax
from jax.experimental import pallas as pl
from jax.experimental.pallas import tpu as pltpu
from jax.experimental.pallas import tpu_sc as plsc

N = 10000
E = 320000
D = 128
DH = 64
NC = 2
NS = 16
NW = NC * NS
STREAM = 128
RPS = 158
EP = NS * RPS * STREAM
ACC_N = 10112
STRIPE = ACC_N // NS
TRASH = N
BLK = 2000
FP = jax.lax.Precision.HIGHEST


def _sc_agg(xh, packed):
    mesh = plsc.VectorSubcoreMesh(core_axis_name="c", subcore_axis_name="s")
    cp = pltpu.CompilerParams(use_tc_tiling_on_sc=False)
    if "needs_layout_passes" in pltpu.CompilerParams.__dataclass_fields__:
        cp = dataclasses.replace(cp, needs_layout_passes=False)

    @functools.partial(
        pl.kernel,
        mesh=mesh,
        compiler_params=cp,
        out_type=jax.ShapeDtypeStruct((NC, ACC_N, DH), jnp.float32),
        scratch_types=[
            pltpu.VMEM((RPS, STREAM), jnp.int32),
            pltpu.VMEM((RPS, STREAM), jnp.int32),
            pltpu.VMEM((RPS, STREAM), jnp.int32),
            pltpu.VMEM((STREAM, DH), jnp.float32),
            pltpu.VMEM((STREAM, DH), jnp.float32),
            pltpu.VMEM((STRIPE // 4, DH), jnp.float32),
            pltpu.VMEM_SHARED((ACC_N, DH), jnp.float32),
            pltpu.SemaphoreType.DMA,
            pltpu.SemaphoreType.DMA,
        ],
    )
    def k(xh_hbm, pk_hbm, p_out, pbuf, sidx, didx, rows_a, rows_b,
          zbuf, acc, sem_a, sem_b):
        c = lax.axis_index("c")
        s = lax.axis_index("s")

        pltpu.sync_copy(pk_hbm.at[s], pbuf)

        @pl.loop(0, RPS)
        def _(j):
            @pl.loop(0, STREAM, step=16)
            def _(kk):
                v = pbuf[j, pl.ds(kk, 16)]
                sidx[j, pl.ds(kk, 16)] = lax.bitwise_and(v, 16383)
                didx[j, pl.ds(kk, 16)] = lax.shift_right_logical(v, 14)

        @pl.loop(0, STRIPE // 4)
        def _(r):
            @pl.loop(0, DH, step=16)
            def _(jj):
                zbuf[r, pl.ds(jj, 16)] = jnp.zeros((16,), jnp.float32)

        for kk in range(4):
            pltpu.sync_copy(
                zbuf, acc.at[pl.ds(s * STRIPE + kk * (STRIPE // 4), STRIPE // 4)])

        plsc.subcore_barrier()

        def gather(j, buf, sem):
            return pltpu.make_async_copy(xh_hbm.at[c].at[sidx.at[j]], buf, sem)

        gather(0, rows_a, sem_a).start()

        @pl.loop(0, RPS, step=2)
        def _(j):
            gather(j, rows_a, sem_a).wait()
            gather(j + 1, rows_b, sem_b).start()
            pltpu.sync_copy(rows_a, acc.at[didx.at[j]], add=True)
            gather(j + 1, rows_b, sem_b).wait()

            @pl.when(j + 2 < RPS)
            def _():
                gather(j + 2, rows_a, sem_a).start()

            pltpu.sync_copy(rows_b, acc.at[didx.at[j + 1]], add=True)

        plsc.subcore_barrier()

        pltpu.sync_copy(
            acc.at[pl.ds(s * STRIPE, STRIPE)],
            p_out.at[c].at[pl.ds(s * STRIPE, STRIPE)],
        )

    return k(xh, packed)


def _sc_deg(packed):
    mesh = plsc.VectorSubcoreMesh(core_axis_name="c", subcore_axis_name="s")
    cp = pltpu.CompilerParams(use_tc_tiling_on_sc=False)
    if "needs_layout_passes" in pltpu.CompilerParams.__dataclass_fields__:
        cp = dataclasses.replace(cp, needs_layout_passes=False)
    half = RPS // 2

    @functools.partial(
        pl.kernel,
        mesh=mesh,
        compiler_params=cp,
        out_type=jax.ShapeDtypeStruct((NW, ACC_N), jnp.float32),
        scratch_types=[
            pltpu.VMEM((half, STREAM), jnp.int32),
            pltpu.VMEM((ACC_N,), jnp.float32),
        ],
    )
    def k(pk_hbm, deg_out, pbuf, degv):
        c = lax.axis_index("c")
        s = lax.axis_index("s")
        w = c * NS + s

        pltpu.sync_copy(pk_hbm.at[s].at[pl.ds(c * half, half)], pbuf)

        @pl.loop(0, ACC_N, step=16)
        def _(i):
            degv[pl.ds(i, 16)] = jnp.zeros((16,), jnp.float32)

        ones16 = jnp.ones((16,), jnp.float32)

        @pl.loop(0, half)
        def _(j):
            @pl.loop(0, STREAM, step=16)
            def _(kk):
                d16 = lax.shift_right_logical(pbuf[j, pl.ds(kk, 16)], 14)
                plsc.addupdate_scatter(degv, [d16], ones16)

        pltpu.sync_copy(degv, deg_out.at[w])

    return k(packed)


def _layer_body(p0_ref, p1_ref, dg_ref, x_ref, wl_ref, bl_ref, wr_ref, out_ref):
    accf = jnp.concatenate([p0_ref[0], p1_ref[0]], axis=1)
    deg = jnp.maximum(jnp.sum(dg_ref[...], axis=1, keepdims=True), 1.0)
    agg = accf / deg
    h = (
        jnp.dot(agg, wl_ref[...], precision=FP)
        + bl_ref[...]
        + jnp.dot(x_ref[...], wr_ref[...], precision=FP)
    )
    out_ref[...] = jnp.maximum(h, 0.0)


def _final_body(p0_ref, p1_ref, dg_ref, h_ref, wl_ref, bl_ref, wr_ref,
                wo_ref, bo_ref, out_ref):
    accf = jnp.concatenate([p0_ref[0], p1_ref[0]], axis=1)
    deg = jnp.maximum(jnp.sum(dg_ref[...], axis=1, keepdims=True), 1.0)
    agg = accf / deg
    h = (
        jnp.dot(agg, wl_ref[...], precision=FP)
        + bl_ref[...]
        + jnp.dot(h_ref[...], wr_ref[...], precision=FP)
    )
    h = jnp.maximum(h, 0.0)
    wod = wo_ref[1:2, :] - wo_ref[0:1, :]
    ld = jnp.sum(h * wod, axis=1, keepdims=True)
    ld = ld + (bo_ref[1] - bo_ref[0])
    out_ref[...] = 1.0 / (1.0 + jnp.exp(-ld))


def _layer_call(p, degT, x, wl, bl, wr):
    return pl.pallas_call(
        _layer_body,
        grid=(N // BLK,),
        in_specs=[
            pl.BlockSpec((1, BLK, DH), lambda i: (0, i, 0)),
            pl.BlockSpec((1, BLK, DH), lambda i: (1, i, 0)),
            pl.BlockSpec((BLK, NW), lambda i: (i, 0)),
            pl.BlockSpec((BLK, D), lambda i: (i, 0)),
            pl.BlockSpec((D, D), lambda i: (0, 0)),
            pl.BlockSpec((1, D), lambda i: (0, 0)),
            pl.BlockSpec((D, D), lambda i: (0, 0)),
        ],
        out_specs=pl.BlockSpec((BLK, D), lambda i: (i, 0)),
        out_shape=jax.ShapeDtypeStruct((N, D), jnp.float32),
    )(p, p, degT, x, wl, bl, wr)


def _final_call(p, degT, h, wl, bl, wr, woT, bo2):
    return pl.pallas_call(
        _final_body,
        grid=(N // BLK,),
        in_specs=[
            pl.BlockSpec((1, BLK, DH), lambda i: (0, i, 0)),
            pl.BlockSpec((1, BLK, DH), lambda i: (1, i, 0)),
            pl.BlockSpec((BLK, NW), lambda i: (i, 0)),
            pl.BlockSpec((BLK, D), lambda i: (i, 0)),
            pl.BlockSpec((D, D), lambda i: (0, 0)),
            pl.BlockSpec((1, D), lambda i: (0, 0)),
            pl.BlockSpec((D, D), lambda i: (0, 0)),
            pl.BlockSpec((2, D), lambda i: (0, 0)),
            pl.BlockSpec(memory_space=pltpu.SMEM),
        ],
        out_specs=pl.BlockSpec((BLK, 1), lambda i: (i, 0)),
        out_shape=jax.ShapeDtypeStruct((N, 1), jnp.float32),
    )(p, p, degT, h, wl, bl, wr, woT, bo2)


def _split_cols(x):
    return jnp.stack([x[:, :DH], x[:, DH:]])


def kernel(x, edge_index, W1l, b1l, W1r, W2l, b2l, W2r, Wo, bo):
    ei = edge_index.astype(jnp.int32)
    npad = EP - E
    src_p = jnp.concatenate([ei[0], jnp.zeros((npad,), jnp.int32)])
    dst_p = jnp.concatenate([ei[1], jnp.full((npad,), TRASH, jnp.int32)])
    packed = ((dst_p << 14) | src_p).reshape(NS, RPS, STREAM)

    b1 = b1l.reshape(1, D)
    b2 = b2l.reshape(1, D)
    woT = Wo.T
    bo2 = bo

    p1 = _sc_agg(_split_cols(x), packed)
    degT = _sc_deg(packed).T[:N]
    h1 = _layer_call(p1[:, :N], degT, x, W1l, b1, W1r)
    p2 = _sc_agg(_split_cols(h1), packed)
    out = _final_call(p2[:, :N], degT, h1, W2l, b2, W2r, woT, bo2)
    return out.reshape(N)


# speedup vs baseline: 5.6220x; 1.0317x over previous
import dataclasses
import functools

import jax
import jax.numpy as jnp
from jax import lax
from jax.experimental import pallas as pl
from jax.experimental.pallas import tpu as pltpu
from jax.experimental.pallas import tpu_sc as plsc

N = 10000
E = 320000
D = 128
DH = 64
NC = 2
NS = 16
NW = NC * NS
STREAM = 128
RPS = 158
EP = NS * RPS * STREAM
ACC_N = 10112
STRIPE = ACC_N // NS
TRASH = N
BLK = 2000
FP = jax.lax.Precision.HIGHEST


def _sc_agg(xh, packed):
    mesh = plsc.VectorSubcoreMesh(core_axis_name="c", subcore_axis_name="s")
    cp = pltpu.CompilerParams(use_tc_tiling_on_sc=False)
    if "needs_layout_passes" in pltpu.CompilerParams.__dataclass_fields__:
        cp = dataclasses.replace(cp, needs_layout_passes=False)

    @functools.partial(
        pl.kernel,
        mesh=mesh,
        compiler_params=cp,
        out_type=jax.ShapeDtypeStruct((NC, ACC_N, DH), jnp.float32),
        scratch_types=[
            pltpu.VMEM((RPS, STREAM), jnp.int32),
            pltpu.VMEM((RPS, STREAM), jnp.int32),
            pltpu.VMEM((RPS, STREAM), jnp.int32),
            pltpu.VMEM((STREAM, DH), jnp.float32),
            pltpu.VMEM((STREAM, DH), jnp.float32),
            pltpu.VMEM((STRIPE // 4, DH), jnp.float32),
            pltpu.VMEM_SHARED((ACC_N, DH), jnp.float32),
            pltpu.SemaphoreType.DMA,
            pltpu.SemaphoreType.DMA,
            pltpu.SemaphoreType.DMA,
            pltpu.SemaphoreType.DMA,
        ],
    )
    def k(xh_hbm, pk_hbm, p_out, pbuf, sidx, didx, rows_a, rows_b,
          zbuf, acc, sem_a, sem_b, sem_sa, sem_sb):
        c = lax.axis_index("c")
        s = lax.axis_index("s")

        pltpu.sync_copy(pk_hbm.at[s], pbuf)

        @pl.loop(0, RPS)
        def _(j):
            @pl.loop(0, STREAM, step=16)
            def _(kk):
                v = pbuf[j, pl.ds(kk, 16)]
                sidx[j, pl.ds(kk, 16)] = lax.bitwise_and(v, 16383)
                didx[j, pl.ds(kk, 16)] = lax.shift_right_logical(v, 14)

        @pl.loop(0, STRIPE // 4)
        def _(r):
            @pl.loop(0, DH, step=16)
            def _(jj):
                zbuf[r, pl.ds(jj, 16)] = jnp.zeros((16,), jnp.float32)

        for kk in range(4):
            pltpu.sync_copy(
                zbuf, acc.at[pl.ds(s * STRIPE + kk * (STRIPE // 4), STRIPE // 4)])

        plsc.subcore_barrier()

        def gather(j, buf, sem):
            return pltpu.make_async_copy(xh_hbm.at[c].at[sidx.at[j]], buf, sem)

        def scatter(j, buf, sem):
            return pltpu.make_async_copy(buf, acc.at[didx.at[j]], sem)

        gather(0, rows_a, sem_a).start()
        gather(1, rows_b, sem_b).start()

        @pl.loop(0, RPS, step=2)
        def _(j):
            gather(j, rows_a, sem_a).wait()
            scatter(j, rows_a, sem_sa).start(add=True)
            gather(j + 1, rows_b, sem_b).wait()
            scatter(j + 1, rows_b, sem_sb).start(add=True)
            scatter(j, rows_a, sem_sa).wait()

            @pl.when(j + 2 < RPS)
            def _():
                gather(j + 2, rows_a, sem_a).start()

            scatter(j + 1, rows_b, sem_sb).wait()

            @pl.when(j + 3 < RPS)
            def _():
                gather(j + 3, rows_b, sem_b).start()

        plsc.subcore_barrier()

        pltpu.sync_copy(
            acc.at[pl.ds(s * STRIPE, STRIPE)],
            p_out.at[c].at[pl.ds(s * STRIPE, STRIPE)],
        )

    return k(xh, packed)


def _sc_deg(packed):
    mesh = plsc.VectorSubcoreMesh(core_axis_name="c", subcore_axis_name="s")
    cp = pltpu.CompilerParams(use_tc_tiling_on_sc=False)
    if "needs_layout_passes" in pltpu.CompilerParams.__dataclass_fields__:
        cp = dataclasses.replace(cp, needs_layout_passes=False)
    half = RPS // 2

    @functools.partial(
        pl.kernel,
        mesh=mesh,
        compiler_params=cp,
        out_type=jax.ShapeDtypeStruct((NW, ACC_N), jnp.float32),
        scratch_types=[
            pltpu.VMEM((half, STREAM), jnp.int32),
            pltpu.VMEM((ACC_N,), jnp.float32),
        ],
    )
    def k(pk_hbm, deg_out, pbuf, degv):
        c = lax.axis_index("c")
        s = lax.axis_index("s")
        w = c * NS + s

        pltpu.sync_copy(pk_hbm.at[s].at[pl.ds(c * half, half)], pbuf)

        @pl.loop(0, ACC_N, step=16)
        def _(i):
            degv[pl.ds(i, 16)] = jnp.zeros((16,), jnp.float32)

        ones16 = jnp.ones((16,), jnp.float32)

        @pl.loop(0, half)
        def _(j):
            @pl.loop(0, STREAM, step=16)
            def _(kk):
                d16 = lax.shift_right_logical(pbuf[j, pl.ds(kk, 16)], 14)
                plsc.addupdate_scatter(degv, [d16], ones16)

        pltpu.sync_copy(degv, deg_out.at[w])

    return k(packed)


def _layer_body(p0_ref, p1_ref, dg_ref, x_ref, wl_ref, bl_ref, wr_ref, out_ref):
    accf = jnp.concatenate([p0_ref[0], p1_ref[0]], axis=1)
    deg = jnp.maximum(jnp.sum(dg_ref[...], axis=1, keepdims=True), 1.0)
    agg = accf / deg
    h = (
        jnp.dot(agg, wl_ref[...], precision=FP)
        + bl_ref[...]
        + jnp.dot(x_ref[...], wr_ref[...], precision=FP)
    )
    out_ref[...] = jnp.maximum(h, 0.0)


def _final_body(p0_ref, p1_ref, dg_ref, h_ref, wl_ref, bl_ref, wr_ref,
                wo_ref, bo_ref, out_ref):
    accf = jnp.concatenate([p0_ref[0], p1_ref[0]], axis=1)
    deg = jnp.maximum(jnp.sum(dg_ref[...], axis=1, keepdims=True), 1.0)
    agg = accf / deg
    h = (
        jnp.dot(agg, wl_ref[...], precision=FP)
        + bl_ref[...]
        + jnp.dot(h_ref[...], wr_ref[...], precision=FP)
    )
    h = jnp.maximum(h, 0.0)
    wod = wo_ref[1:2, :] - wo_ref[0:1, :]
    ld = jnp.sum(h * wod, axis=1, keepdims=True)
    ld = ld + (bo_ref[1] - bo_ref[0])
    out_ref[...] = 1.0 / (1.0 + jnp.exp(-ld))


def _layer_call(p, degT, x, wl, bl, wr):
    return pl.pallas_call(
        _layer_body,
        grid=(N // BLK,),
        in_specs=[
            pl.BlockSpec((1, BLK, DH), lambda i: (0, i, 0)),
            pl.BlockSpec((1, BLK, DH), lambda i: (1, i, 0)),
            pl.BlockSpec((BLK, NW), lambda i: (i, 0)),
            pl.BlockSpec((BLK, D), lambda i: (i, 0)),
            pl.BlockSpec((D, D), lambda i: (0, 0)),
            pl.BlockSpec((1, D), lambda i: (0, 0)),
            pl.BlockSpec((D, D), lambda i: (0, 0)),
        ],
        out_specs=pl.BlockSpec((BLK, D), lambda i: (i, 0)),
        out_shape=jax.ShapeDtypeStruct((N, D), jnp.float32),
    )(p, p, degT, x, wl, bl, wr)


def _final_call(p, degT, h, wl, bl, wr, woT, bo2):
    return pl.pallas_call(
        _final_body,
        grid=(N // BLK,),
        in_specs=[
            pl.BlockSpec((1, BLK, DH), lambda i: (0, i, 0)),
            pl.BlockSpec((1, BLK, DH), lambda i: (1, i, 0)),
            pl.BlockSpec((BLK, NW), lambda i: (i, 0)),
            pl.BlockSpec((BLK, D), lambda i: (i, 0)),
            pl.BlockSpec((D, D), lambda i: (0, 0)),
            pl.BlockSpec((1, D), lambda i: (0, 0)),
            pl.BlockSpec((D, D), lambda i: (0, 0)),
            pl.BlockSpec((2, D), lambda i: (0, 0)),
            pl.BlockSpec(memory_space=pltpu.SMEM),
        ],
        out_specs=pl.BlockSpec((BLK, 1), lambda i: (i, 0)),
        out_shape=jax.ShapeDtypeStruct((N, 1), jnp.float32),
    )(p, p, degT, h, wl, bl, wr, woT, bo2)


def _split_cols(x):
    return jnp.stack([x[:, :DH], x[:, DH:]])


def kernel(x, edge_index, W1l, b1l, W1r, W2l, b2l, W2r, Wo, bo):
    ei = edge_index.astype(jnp.int32)
    npad = EP - E
    src_p = jnp.concatenate([ei[0], jnp.zeros((npad,), jnp.int32)])
    dst_p = jnp.concatenate([ei[1], jnp.full((npad,), TRASH, jnp.int32)])
    packed = ((dst_p << 14) | src_p).reshape(NS, RPS, STREAM)

    b1 = b1l.reshape(1, D)
    b2 = b2l.reshape(1, D)
    woT = Wo.T
    bo2 = bo

    p1 = _sc_agg(_split_cols(x), packed)
    degT = _sc_deg(packed).T[:N]
    h1 = _layer_call(p1[:, :N], degT, x, W1l, b1, W1r)
    p2 = _sc_agg(_split_cols(h1), packed)
    out = _final_call(p2[:, :N], degT, h1, W2l, b2, W2r, woT, bo2)
    return out.reshape(N)


# speedup vs baseline: 6.0971x; 1.0845x over previous
import dataclasses
import functools

import jax
import jax.numpy as jnp
from jax import lax
from jax.experimental import pallas as pl
from jax.experimental.pallas import tpu as pltpu
from jax.experimental.pallas import tpu_sc as plsc

N = 10000
E = 320000
D = 128
DH = 64
NC = 2
NS = 16
NW = NC * NS
STREAM = 128
RPS = 158
EP = NS * RPS * STREAM
ACC_N = 10112
STRIPE = ACC_N // NS
TRASH = N
BLK = 2000
FP = jax.lax.Precision.DEFAULT


def _sc_agg(xh, packed):
    mesh = plsc.VectorSubcoreMesh(core_axis_name="c", subcore_axis_name="s")
    cp = pltpu.CompilerParams(use_tc_tiling_on_sc=False)
    if "needs_layout_passes" in pltpu.CompilerParams.__dataclass_fields__:
        cp = dataclasses.replace(cp, needs_layout_passes=False)

    @functools.partial(
        pl.kernel,
        mesh=mesh,
        compiler_params=cp,
        out_type=jax.ShapeDtypeStruct((NC, ACC_N, DH), jnp.float32),
        scratch_types=[
            pltpu.VMEM((RPS, STREAM), jnp.int32),
            pltpu.VMEM((RPS, STREAM), jnp.int32),
            pltpu.VMEM((RPS, STREAM), jnp.int32),
            pltpu.VMEM((STREAM, DH), jnp.float32),
            pltpu.VMEM((STREAM, DH), jnp.float32),
            pltpu.VMEM((STRIPE // 4, DH), jnp.float32),
            pltpu.VMEM_SHARED((ACC_N, DH), jnp.float32),
            pltpu.SemaphoreType.DMA,
            pltpu.SemaphoreType.DMA,
            pltpu.SemaphoreType.DMA,
            pltpu.SemaphoreType.DMA,
        ],
    )
    def k(xh_hbm, pk_hbm, p_out, pbuf, sidx, didx, rows_a, rows_b,
          zbuf, acc, sem_a, sem_b, sem_sa, sem_sb):
        c = lax.axis_index("c")
        s = lax.axis_index("s")

        pltpu.sync_copy(pk_hbm.at[s], pbuf)

        @pl.loop(0, RPS)
        def _(j):
            @pl.loop(0, STREAM, step=16)
            def _(kk):
                v = pbuf[j, pl.ds(kk, 16)]
                sidx[j, pl.ds(kk, 16)] = lax.bitwise_and(v, 16383)
                didx[j, pl.ds(kk, 16)] = lax.shift_right_logical(v, 14)

        @pl.loop(0, STRIPE // 4)
        def _(r):
            @pl.loop(0, DH, step=16)
            def _(jj):
                zbuf[r, pl.ds(jj, 16)] = jnp.zeros((16,), jnp.float32)

        for kk in range(4):
            pltpu.sync_copy(
                zbuf, acc.at[pl.ds(s * STRIPE + kk * (STRIPE // 4), STRIPE // 4)])

        plsc.subcore_barrier()

        def gather(j, buf, sem):
            return pltpu.make_async_copy(xh_hbm.at[c].at[sidx.at[j]], buf, sem)

        def scatter(j, buf, sem):
            return pltpu.make_async_copy(buf, acc.at[didx.at[j]], sem)

        gather(0, rows_a, sem_a).start()
        gather(1, rows_b, sem_b).start()

        @pl.loop(0, RPS, step=2)
        def _(j):
            gather(j, rows_a, sem_a).wait()
            scatter(j, rows_a, sem_sa).start(add=True)
            gather(j + 1, rows_b, sem_b).wait()
            scatter(j + 1, rows_b, sem_sb).start(add=True)
            scatter(j, rows_a, sem_sa).wait()

            @pl.when(j + 2 < RPS)
            def _():
                gather(j + 2, rows_a, sem_a).start()

            scatter(j + 1, rows_b, sem_sb).wait()

            @pl.when(j + 3 < RPS)
            def _():
                gather(j + 3, rows_b, sem_b).start()

        plsc.subcore_barrier()

        pltpu.sync_copy(
            acc.at[pl.ds(s * STRIPE, STRIPE)],
            p_out.at[c].at[pl.ds(s * STRIPE, STRIPE)],
        )

    return k(xh, packed)


def _sc_deg(packed):
    mesh = plsc.VectorSubcoreMesh(core_axis_name="c", subcore_axis_name="s")
    cp = pltpu.CompilerParams(use_tc_tiling_on_sc=False)
    if "needs_layout_passes" in pltpu.CompilerParams.__dataclass_fields__:
        cp = dataclasses.replace(cp, needs_layout_passes=False)
    half = RPS // 2

    @functools.partial(
        pl.kernel,
        mesh=mesh,
        compiler_params=cp,
        out_type=jax.ShapeDtypeStruct((NW, ACC_N), jnp.float32),
        scratch_types=[
            pltpu.VMEM((half, STREAM), jnp.int32),
            pltpu.VMEM((ACC_N,), jnp.float32),
        ],
    )
    def k(pk_hbm, deg_out, pbuf, degv):
        c = lax.axis_index("c")
        s = lax.axis_index("s")
        w = c * NS + s

        pltpu.sync_copy(pk_hbm.at[s].at[pl.ds(c * half, half)], pbuf)

        @pl.loop(0, ACC_N, step=16)
        def _(i):
            degv[pl.ds(i, 16)] = jnp.zeros((16,), jnp.float32)

        ones16 = jnp.ones((16,), jnp.float32)

        @pl.loop(0, half)
        def _(j):
            @pl.loop(0, STREAM, step=16)
            def _(kk):
                d16 = lax.shift_right_logical(pbuf[j, pl.ds(kk, 16)], 14)
                plsc.addupdate_scatter(degv, [d16], ones16)

        pltpu.sync_copy(degv, deg_out.at[w])

    return k(packed)


def _mm_body(x_ref, w_ref, b_ref, out_ref):
    out_ref[...] = jnp.dot(x_ref[...], w_ref[...], precision=FP) + b_ref[...]


def _mm_call(x, w, b):
    return pl.pallas_call(
        _mm_body,
        grid=(N // BLK,),
        in_specs=[
            pl.BlockSpec((BLK, D), lambda i: (i, 0)),
            pl.BlockSpec((D, D), lambda i: (0, 0)),
            pl.BlockSpec((1, D), lambda i: (0, 0)),
        ],
        out_specs=pl.BlockSpec((BLK, D), lambda i: (i, 0)),
        out_shape=jax.ShapeDtypeStruct((N, D), jnp.float32),
    )(x, w, b)


def _mm_split_body(h0_ref, h1_ref, w_ref, b_ref, out_ref):
    out_ref[...] = (
        jnp.dot(h0_ref[0], w_ref[:DH], precision=FP)
        + jnp.dot(h1_ref[0], w_ref[DH:], precision=FP)
        + b_ref[...]
    )


def _mm_split_call(hs, w, b):
    return pl.pallas_call(
        _mm_split_body,
        grid=(N // BLK,),
        in_specs=[
            pl.BlockSpec((1, BLK, DH), lambda i: (0, i, 0)),
            pl.BlockSpec((1, BLK, DH), lambda i: (1, i, 0)),
            pl.BlockSpec((D, D), lambda i: (0, 0)),
            pl.BlockSpec((1, D), lambda i: (0, 0)),
        ],
        out_specs=pl.BlockSpec((BLK, D), lambda i: (i, 0)),
        out_shape=jax.ShapeDtypeStruct((N, D), jnp.float32),
    )(hs, hs, w, b)


def _layer_body(p0_ref, p1_ref, dg_ref, xr_ref, wl_ref, out_ref):
    accf = jnp.concatenate([p0_ref[0], p1_ref[0]], axis=1)
    deg = jnp.maximum(jnp.sum(dg_ref[...], axis=1, keepdims=True), 1.0)
    agg = accf / deg
    h = jnp.maximum(jnp.dot(agg, wl_ref[...], precision=FP) + xr_ref[...], 0.0)
    out_ref[0] = h[:, :DH]
    out_ref[1] = h[:, DH:]


def _final_body(p0_ref, p1_ref, dg_ref, xr_ref, wl_ref, wo_ref, bo_ref,
                out_ref):
    accf = jnp.concatenate([p0_ref[0], p1_ref[0]], axis=1)
    deg = jnp.maximum(jnp.sum(dg_ref[...], axis=1, keepdims=True), 1.0)
    agg = accf / deg
    h = jnp.maximum(jnp.dot(agg, wl_ref[...], precision=FP) + xr_ref[...], 0.0)
    wod = wo_ref[1:2, :] - wo_ref[0:1, :]
    ld = jnp.sum(h * wod, axis=1, keepdims=True)
    ld = ld + (bo_ref[1] - bo_ref[0])
    out_ref[...] = 1.0 / (1.0 + jnp.exp(-ld))


def _layer_call(p, degT, xr, wl):
    return pl.pallas_call(
        _layer_body,
        grid=(N // BLK,),
        in_specs=[
            pl.BlockSpec((1, BLK, DH), lambda i: (0, i, 0)),
            pl.BlockSpec((1, BLK, DH), lambda i: (1, i, 0)),
            pl.BlockSpec((BLK, NW), lambda i: (i, 0)),
            pl.BlockSpec((BLK, D), lambda i: (i, 0)),
            pl.BlockSpec((D, D), lambda i: (0, 0)),
        ],
        out_specs=pl.BlockSpec((NC, BLK, DH), lambda i: (0, i, 0)),
        out_shape=jax.ShapeDtypeStruct((NC, N, DH), jnp.float32),
    )(p, p, degT, xr, wl)


def _final_call(p, degT, xr, wl, woT, bo2):
    return pl.pallas_call(
        _final_body,
        grid=(N // BLK,),
        in_specs=[
            pl.BlockSpec((1, BLK, DH), lambda i: (0, i, 0)),
            pl.BlockSpec((1, BLK, DH), lambda i: (1, i, 0)),
            pl.BlockSpec((BLK, NW), lambda i: (i, 0)),
            pl.BlockSpec((BLK, D), lambda i: (i, 0)),
            pl.BlockSpec((D, D), lambda i: (0, 0)),
            pl.BlockSpec((2, D), lambda i: (0, 0)),
            pl.BlockSpec(memory_space=pltpu.SMEM),
        ],
        out_specs=pl.BlockSpec((BLK, 1), lambda i: (i, 0)),
        out_shape=jax.ShapeDtypeStruct((N, 1), jnp.float32),
    )(p, p, degT, xr, wl, woT, bo2)


def _split_cols(x):
    return jnp.stack([x[:, :DH], x[:, DH:]])


def kernel(x, edge_index, W1l, b1l, W1r, W2l, b2l, W2r, Wo, bo):
    ei = edge_index.astype(jnp.int32)
    npad = EP - E
    src_p = jnp.concatenate([ei[0], jnp.zeros((npad,), jnp.int32)])
    dst_p = jnp.concatenate([ei[1], jnp.full((npad,), TRASH, jnp.int32)])
    packed = ((dst_p << 14) | src_p).reshape(NS, RPS, STREAM)

    b1 = b1l.reshape(1, D)
    b2 = b2l.reshape(1, D)
    woT = Wo.T
    bo2 = bo

    p1 = _sc_agg(_split_cols(x), packed)
    degs = _sc_deg(packed)
    xr1 = _mm_call(x, W1r, b1)
    degT = degs.T
    h1s = _layer_call(p1, degT, xr1, W1l)
    p2 = _sc_agg(h1s, packed)
    xr2 = _mm_split_call(h1s, W2r, b2)
    out = _final_call(p2, degT, xr2, W2l, woT, bo2)
    return out.reshape(N)
